# baseline (speedup 1.0000x reference)
import functools

import jax
import jax.numpy as jnp
from jax import lax
from jax.experimental import pallas as pl
from jax.experimental.pallas import tpu as pltpu
from jax.experimental.pallas import tpu_sc as plsc

N = 10000
E = 320000
H = 128

NC = 2
NS = 16
NW = NC * NS
EPW = E // NW
CHUNK = 128
NCHUNK = 80
EPAD = NCHUNK * CHUNK
NHALF = 2
HCHUNK = NCHUNK // NHALF
HEP = HCHUNK * CHUNK
NPAIR = (HCHUNK - 2) // 2
NP = 10240
RPT = NP // NS
ZROWS = RPT


def _sc_agg_body(with_counts, t_hbm, src_hbm, dst_hbm, zrow_hbm, zcnt_hbm,
                 one_hbm, *rest):
    if with_counts:
        (out_sum, out_cnt, acc, acc_cnt, srcv, dstv, onev, rows_a, rows_b,
         gs_a, gs_b) = rest
    else:
        (out_sum, acc, srcv, dstv, rows_a, rows_b, gs_a, gs_b) = rest
    cid = lax.axis_index("c")
    sid = lax.axis_index("s")
    wid = cid * NS + sid

    pltpu.sync_copy(zrow_hbm, acc.at[pl.ds(sid * RPT, RPT), :])
    if with_counts:
        pltpu.sync_copy(zcnt_hbm, acc_cnt.at[pl.ds(sid * RPT, RPT)])
        pltpu.sync_copy(one_hbm, onev)
    plsc.subcore_barrier()

    def gather(c, buf, sem):
        idx = srcv.at[pl.ds(c * CHUNK, CHUNK)]
        return pltpu.async_copy(t_hbm.at[idx], buf, sem)

    def gwait(buf, sem):
        pltpu.make_async_copy(t_hbm.at[srcv.at[pl.ds(0, CHUNK)]],
                              buf, sem).wait()

    def scat(c, buf):
        pltpu.sync_copy(buf, acc.at[dstv.at[c]], add=True)
        if with_counts:
            pltpu.sync_copy(onev, acc_cnt.at[dstv.at[c]], add=True)

    def pair(g, carry):
        c0 = g * 2
        gather(c0 + 1, rows_b, gs_b)
        gwait(rows_a, gs_a)
        scat(c0, rows_a)
        gather(c0 + 2, rows_a, gs_a)
        gwait(rows_b, gs_b)
        scat(c0 + 1, rows_b)
        return carry

    for h in range(NHALF):
        pltpu.sync_copy(
            src_hbm.at[pl.ds(pl.multiple_of(wid * EPAD + h * HEP, 8), HEP)],
            srcv)
        pltpu.sync_copy(dst_hbm.at[wid, pl.ds(h * HCHUNK, HCHUNK)], dstv)
        pltpu.async_copy(t_hbm.at[srcv.at[pl.ds(0, CHUNK)]], rows_a, gs_a)
        lax.fori_loop(0, NPAIR, pair, 0)
        gather(HCHUNK - 1, rows_b, gs_b)
        gwait(rows_a, gs_a)
        scat(HCHUNK - 2, rows_a)
        gwait(rows_b, gs_b)
        scat(HCHUNK - 1, rows_b)
    plsc.subcore_barrier()

    pltpu.sync_copy(acc.at[pl.ds(sid * RPT, RPT), :],
                    out_sum.at[cid, pl.ds(sid * RPT, RPT), :])
    if with_counts:
        pltpu.sync_copy(acc_cnt.at[pl.ds(sid * RPT, RPT)],
                        out_cnt.at[cid, pl.ds(sid * RPT, RPT)])


_SUM_T = jax.ShapeDtypeStruct((NC, NP, H), jnp.float32)
_CNT_T = jax.ShapeDtypeStruct((NC, NP), jnp.float32)

_SCRATCH_COMMON = dict(
    acc=pltpu.VMEM_SHARED((NP, H), jnp.float32),
    srcv=pltpu.VMEM((HEP,), jnp.int32),
    dstv=pltpu.VMEM((HCHUNK, CHUNK), jnp.int32),
    rows_a=pltpu.VMEM((CHUNK, H), jnp.float32),
    rows_b=pltpu.VMEM((CHUNK, H), jnp.float32),
)

_SCRATCH_C = [
    _SCRATCH_COMMON["acc"],
    pltpu.VMEM_SHARED((NP,), jnp.float32),
    _SCRATCH_COMMON["srcv"],
    _SCRATCH_COMMON["dstv"],
    pltpu.VMEM((CHUNK,), jnp.float32),
    _SCRATCH_COMMON["rows_a"],
    _SCRATCH_COMMON["rows_b"],
    pltpu.SemaphoreType.DMA,
    pltpu.SemaphoreType.DMA,
]

@functools.cache
def _get_sc_aggs():
    mesh = plsc.VectorSubcoreMesh(
        core_axis_name="c", subcore_axis_name="s",
        num_cores=NC, num_subcores=NS)
    agg_c = pl.kernel(functools.partial(_sc_agg_body, True),
                      out_type=(_SUM_T, _CNT_T), mesh=mesh,
                      scratch_types=_SCRATCH_C)
    return agg_c, agg_c


def _pre_body(x_ref, w_ref, b_ref, o_ref):
    v = jnp.dot(x_ref[...], w_ref[...],
                preferred_element_type=jnp.float32) + b_ref[...]
    o_ref[...] = jnp.where(v >= 0, v, 0.01 * v)


def _combine_body(ps_ref, pc_ref, h_ref, wl_ref, bl_ref, wr_ref, o_ref):
    s = ps_ref[0, :N] + ps_ref[1, :N]
    c = jnp.maximum(pc_ref[0, :N] + pc_ref[1, :N], 1.0)
    agg = s / c
    v = (jnp.dot(agg, wl_ref[...], preferred_element_type=jnp.float32)
         + bl_ref[...]
         + jnp.dot(h_ref[...], wr_ref[...], preferred_element_type=jnp.float32))
    o_ref[...] = jnp.where(v >= 0, v, 0.01 * v) + h_ref[...]


def _combine_post_body(ps_ref, pc_ref, h_ref, wl_ref, bl_ref, wr_ref,
                       wp_ref, bp_ref, o_ref):
    s = ps_ref[0, :N] + ps_ref[1, :N]
    c = jnp.maximum(pc_ref[0, :N] + pc_ref[1, :N], 1.0)
    agg = s / c
    v = (jnp.dot(agg, wl_ref[...], preferred_element_type=jnp.float32)
         + bl_ref[...]
         + jnp.dot(h_ref[...], wr_ref[...], preferred_element_type=jnp.float32))
    h2 = jnp.where(v >= 0, v, 0.01 * v) + h_ref[...]
    o_ref[...] = jnp.dot(h2, wp_ref[...],
                         preferred_element_type=jnp.float32) + bp_ref[...]


_NH = jax.ShapeDtypeStruct((N, H), jnp.float32)

_tc_pre = pl.pallas_call(_pre_body, out_shape=_NH)
_tc_combine = pl.pallas_call(_combine_body, out_shape=_NH)
_tc_combine_post = pl.pallas_call(_combine_post_body, out_shape=_NH)


def kernel(x, edge_index, W_pre, b_pre, Wl0, bl0, Wr0, Wl1, bl1, Wr1,
           W_post, b_post):
    pad = EPAD - EPW
    src = jnp.pad(edge_index[0].reshape(NW, EPW),
                  ((0, 0), (0, pad))).reshape(-1)
    dst = jnp.pad(edge_index[1].reshape(NW, EPW), ((0, 0), (0, pad)),
                  constant_values=NP - 1).reshape(NW, NCHUNK, CHUNK)
    zrow = jnp.zeros((RPT, H), jnp.float32)
    zcnt = jnp.zeros((RPT,), jnp.float32)
    ones = jnp.ones((CHUNK,), jnp.float32)

    agg_c, agg_n = _get_sc_aggs()
    h0 = _tc_pre(x, W_pre, b_pre.reshape(1, H))
    ps0, pc0 = agg_c(h0, src, dst, zrow, zcnt, ones)
    pc = pc0.reshape(NC, NP, 1)
    h1 = _tc_combine(ps0, pc, h0, Wl0, bl0.reshape(1, H), Wr0)
    ps1, _ = agg_n(h1, src, dst, zrow, zcnt, ones)
    return _tc_combine_post(ps1, pc, h1, Wl1, bl1.reshape(1, H), Wr1,
                            W_post, b_post.reshape(1, H))


# speedup vs baseline: 2.6985x; 2.6985x over previous
import functools

import jax
import jax.numpy as jnp
from jax import lax
from jax.experimental import pallas as pl
from jax.experimental.pallas import tpu as pltpu
from jax.experimental.pallas import tpu_sc as plsc

N = 10000
E = 320000
H = 128

NC = 2
NS = 16
NW = NC * NS
EPW = E // NW
CHUNK = 80
NCHUNK = EPW // CHUNK
NPAIR = NCHUNK // 2
NP = 10240
RPT = NP // NS
ZROWS = RPT


def _sc_agg_body(with_counts, t_hbm, src_hbm, dst_hbm, zrow_hbm, zcnt_hbm,
                 one_hbm, *rest):
    if with_counts:
        (out_sum, out_cnt, acc, acc_cnt, srcv, dstv, onev, rows_a, rows_b,
         gs_a, gs_b) = rest
    else:
        (out_sum, acc, srcv, dstv, rows_a, rows_b, gs_a, gs_b) = rest
    cid = lax.axis_index("c")
    sid = lax.axis_index("s")
    wid = cid * NS + sid

    pltpu.sync_copy(zrow_hbm, acc.at[pl.ds(sid * RPT, RPT), :])
    if with_counts:
        pltpu.sync_copy(zcnt_hbm, acc_cnt.at[pl.ds(sid * RPT, RPT)])
        pltpu.sync_copy(one_hbm, onev)
    plsc.subcore_barrier()

    def gather(c, buf, sem):
        idx = srcv.at[pl.ds(c * CHUNK, CHUNK)]
        return pltpu.async_copy(t_hbm.at[idx], buf, sem)

    def gwait(buf, sem):
        pltpu.make_async_copy(t_hbm.at[srcv.at[pl.ds(0, CHUNK)]],
                              buf, sem).wait()

    def scat(c, buf):
        pltpu.sync_copy(buf, acc.at[dstv.at[c]], add=True)
        if with_counts:
            pltpu.sync_copy(onev, acc_cnt.at[dstv.at[c]], add=True)

    def pair(g, carry):
        c0 = g * 2
        gather(c0 + 1, rows_b, gs_b)
        gwait(rows_a, gs_a)
        scat(c0, rows_a)
        gather(c0 + 2, rows_a, gs_a)
        gwait(rows_b, gs_b)
        scat(c0 + 1, rows_b)
        return carry

    pltpu.sync_copy(src_hbm.at[pl.ds(pl.multiple_of(wid * EPW, 8), EPW)], srcv)
    pltpu.sync_copy(dst_hbm.at[wid], dstv)
    pltpu.async_copy(t_hbm.at[srcv.at[pl.ds(0, CHUNK)]], rows_a, gs_a)
    lax.fori_loop(0, NPAIR, pair, 0)
    gwait(rows_a, gs_a)
    scat(NCHUNK - 1, rows_a)
    plsc.subcore_barrier()

    pltpu.sync_copy(acc.at[pl.ds(sid * RPT, RPT), :],
                    out_sum.at[cid, pl.ds(sid * RPT, RPT), :])
    if with_counts:
        pltpu.sync_copy(acc_cnt.at[pl.ds(sid * RPT, RPT)],
                        out_cnt.at[cid, pl.ds(sid * RPT, RPT)])


_SUM_T = jax.ShapeDtypeStruct((NC, NP, H), jnp.float32)
_CNT_T = jax.ShapeDtypeStruct((NC, NP), jnp.float32)

_SCRATCH_COMMON = dict(
    acc=pltpu.VMEM_SHARED((NP, H), jnp.float32),
    srcv=pltpu.VMEM((EPW,), jnp.int32),
    dstv=pltpu.VMEM((NCHUNK, CHUNK), jnp.int32),
    rows_a=pltpu.VMEM((CHUNK, H), jnp.float32),
    rows_b=pltpu.VMEM((CHUNK, H), jnp.float32),
)

_SCRATCH_C = [
    _SCRATCH_COMMON["acc"],
    pltpu.VMEM_SHARED((NP,), jnp.float32),
    _SCRATCH_COMMON["srcv"],
    _SCRATCH_COMMON["dstv"],
    pltpu.VMEM((CHUNK,), jnp.float32),
    _SCRATCH_COMMON["rows_a"],
    _SCRATCH_COMMON["rows_b"],
    pltpu.SemaphoreType.DMA,
    pltpu.SemaphoreType.DMA,
]

@functools.cache
def _get_sc_aggs():
    mesh = plsc.VectorSubcoreMesh(
        core_axis_name="c", subcore_axis_name="s",
        num_cores=NC, num_subcores=NS)
    agg_c = pl.kernel(functools.partial(_sc_agg_body, True),
                      out_type=(_SUM_T, _CNT_T), mesh=mesh,
                      scratch_types=_SCRATCH_C)
    return agg_c, agg_c


def _pre_body(x_ref, w_ref, b_ref, o_ref):
    v = jnp.dot(x_ref[...], w_ref[...],
                preferred_element_type=jnp.float32) + b_ref[...]
    o_ref[...] = jnp.where(v >= 0, v, 0.01 * v)


def _combine_body(ps_ref, pc_ref, h_ref, wl_ref, bl_ref, wr_ref, o_ref):
    s = ps_ref[0, :N] + ps_ref[1, :N]
    c = jnp.maximum(pc_ref[0, :N] + pc_ref[1, :N], 1.0)
    agg = s / c
    v = (jnp.dot(agg, wl_ref[...], preferred_element_type=jnp.float32)
         + bl_ref[...]
         + jnp.dot(h_ref[...], wr_ref[...], preferred_element_type=jnp.float32))
    o_ref[...] = jnp.where(v >= 0, v, 0.01 * v) + h_ref[...]


def _combine_post_body(ps_ref, pc_ref, h_ref, wl_ref, bl_ref, wr_ref,
                       wp_ref, bp_ref, o_ref):
    s = ps_ref[0, :N] + ps_ref[1, :N]
    c = jnp.maximum(pc_ref[0, :N] + pc_ref[1, :N], 1.0)
    agg = s / c
    v = (jnp.dot(agg, wl_ref[...], preferred_element_type=jnp.float32)
         + bl_ref[...]
         + jnp.dot(h_ref[...], wr_ref[...], preferred_element_type=jnp.float32))
    h2 = jnp.where(v >= 0, v, 0.01 * v) + h_ref[...]
    o_ref[...] = jnp.dot(h2, wp_ref[...],
                         preferred_element_type=jnp.float32) + bp_ref[...]


_NH = jax.ShapeDtypeStruct((N, H), jnp.float32)

_tc_pre = pl.pallas_call(_pre_body, out_shape=_NH)
_tc_combine = pl.pallas_call(_combine_body, out_shape=_NH)
_tc_combine_post = pl.pallas_call(_combine_post_body, out_shape=_NH)


def kernel(x, edge_index, W_pre, b_pre, Wl0, bl0, Wr0, Wl1, bl1, Wr1,
           W_post, b_post):
    src = edge_index[0]
    dst = edge_index[1].reshape(NW, NCHUNK, CHUNK)
    zrow = jnp.zeros((RPT, H), jnp.float32)
    zcnt = jnp.zeros((RPT,), jnp.float32)
    ones = jnp.ones((CHUNK,), jnp.float32)

    agg_c, agg_n = _get_sc_aggs()
    h0 = _tc_pre(x, W_pre, b_pre.reshape(1, H))
    ps0, pc0 = agg_c(h0, src, dst, zrow, zcnt, ones)
    pc = pc0.reshape(NC, NP, 1)
    h1 = _tc_combine(ps0, pc, h0, Wl0, bl0.reshape(1, H), Wr0)
    ps1, _ = agg_n(h1, src, dst, zrow, zcnt, ones)
    return _tc_combine_post(ps1, pc, h1, Wl1, bl1.reshape(1, H), Wr1,
                            W_post, b_post.reshape(1, H))


# speedup vs baseline: 2.7605x; 1.0230x over previous
import functools

import jax
import jax.numpy as jnp
from jax import lax
from jax.experimental import pallas as pl
from jax.experimental.pallas import tpu as pltpu
from jax.experimental.pallas import tpu_sc as plsc

N = 10000
E = 320000
H = 128

NC = 2
NS = 16
NW = NC * NS
EPW = E // NW
CHUNK = 80
NCHUNK = EPW // CHUNK
NPAIR = NCHUNK // 2
NP = 10240
RPT = NP // NS
ZROWS = RPT


def _sc_agg_body(with_counts, t_hbm, src_hbm, dst_hbm, zrow_hbm, zcnt_hbm,
                 one_hbm, *rest):
    if with_counts:
        (out_sum, out_cnt, acc, acc_cnt, srcv, dstv, onev, rows_a, rows_b,
         gs_a, gs_b, gs_z, gs_i) = rest
    else:
        (out_sum, acc, srcv, dstv, rows_a, rows_b, gs_a, gs_b, gs_z,
         gs_i) = rest
    cid = lax.axis_index("c")
    sid = lax.axis_index("s")
    wid = cid * NS + sid

    zrow_dst = acc.at[pl.ds(sid * RPT, RPT), :]
    pltpu.async_copy(zrow_hbm, zrow_dst, gs_z)
    zcnt_dst = None
    if with_counts:
        zcnt_dst = acc_cnt.at[pl.ds(sid * RPT, RPT)]
        pltpu.async_copy(zcnt_hbm, zcnt_dst, gs_z)
        pltpu.sync_copy(one_hbm, onev)

    src_slab = src_hbm.at[pl.ds(pl.multiple_of(wid * EPW, 8), EPW)]
    pltpu.async_copy(src_slab, srcv, gs_i)
    pltpu.async_copy(dst_hbm.at[wid], dstv, gs_i)
    pltpu.make_async_copy(src_slab, srcv, gs_i).wait()
    pltpu.make_async_copy(dst_hbm.at[wid], dstv, gs_i).wait()
    pltpu.async_copy(t_hbm.at[srcv.at[pl.ds(0, CHUNK)]], rows_a, gs_a)

    pltpu.make_async_copy(zrow_hbm, zrow_dst, gs_z).wait()
    if with_counts:
        pltpu.make_async_copy(zcnt_hbm, zcnt_dst, gs_z).wait()
    plsc.subcore_barrier()

    def gather(c, buf, sem):
        idx = srcv.at[pl.ds(c * CHUNK, CHUNK)]
        return pltpu.async_copy(t_hbm.at[idx], buf, sem)

    def gwait(buf, sem):
        pltpu.make_async_copy(t_hbm.at[srcv.at[pl.ds(0, CHUNK)]],
                              buf, sem).wait()

    def scat(c, buf):
        pltpu.sync_copy(buf, acc.at[dstv.at[c]], add=True)
        if with_counts:
            pltpu.sync_copy(onev, acc_cnt.at[dstv.at[c]], add=True)

    def pair(g, carry):
        c0 = g * 2
        gather(c0 + 1, rows_b, gs_b)
        gwait(rows_a, gs_a)
        scat(c0, rows_a)
        gather(c0 + 2, rows_a, gs_a)
        gwait(rows_b, gs_b)
        scat(c0 + 1, rows_b)
        return carry

    lax.fori_loop(0, NPAIR, pair, 0)
    gwait(rows_a, gs_a)
    scat(NCHUNK - 1, rows_a)
    plsc.subcore_barrier()

    pltpu.sync_copy(acc.at[pl.ds(sid * RPT, RPT), :],
                    out_sum.at[cid, pl.ds(sid * RPT, RPT), :])
    if with_counts:
        pltpu.sync_copy(acc_cnt.at[pl.ds(sid * RPT, RPT)],
                        out_cnt.at[cid, pl.ds(sid * RPT, RPT)])


_SUM_T = jax.ShapeDtypeStruct((NC, NP, H), jnp.float32)
_CNT_T = jax.ShapeDtypeStruct((NC, NP), jnp.float32)

_SCRATCH_COMMON = dict(
    acc=pltpu.VMEM_SHARED((NP, H), jnp.float32),
    srcv=pltpu.VMEM((EPW,), jnp.int32),
    dstv=pltpu.VMEM((NCHUNK, CHUNK), jnp.int32),
    rows_a=pltpu.VMEM((CHUNK, H), jnp.float32),
    rows_b=pltpu.VMEM((CHUNK, H), jnp.float32),
)

_SCRATCH_C = [
    _SCRATCH_COMMON["acc"],
    pltpu.VMEM_SHARED((NP,), jnp.float32),
    _SCRATCH_COMMON["srcv"],
    _SCRATCH_COMMON["dstv"],
    pltpu.VMEM((CHUNK,), jnp.float32),
    _SCRATCH_COMMON["rows_a"],
    _SCRATCH_COMMON["rows_b"],
    pltpu.SemaphoreType.DMA,
    pltpu.SemaphoreType.DMA,
    pltpu.SemaphoreType.DMA,
    pltpu.SemaphoreType.DMA,
]

@functools.cache
def _get_sc_aggs():
    mesh = plsc.VectorSubcoreMesh(
        core_axis_name="c", subcore_axis_name="s",
        num_cores=NC, num_subcores=NS)
    agg_c = pl.kernel(functools.partial(_sc_agg_body, True),
                      out_type=(_SUM_T, _CNT_T), mesh=mesh,
                      scratch_types=_SCRATCH_C)
    return agg_c, agg_c


def _pre_body(x_ref, w_ref, b_ref, o_ref):
    v = jnp.dot(x_ref[...], w_ref[...],
                preferred_element_type=jnp.float32) + b_ref[...]
    o_ref[...] = jnp.where(v >= 0, v, 0.01 * v)


def _combine_body(ps_ref, pc_ref, h_ref, wl_ref, bl_ref, wr_ref, o_ref):
    s = ps_ref[0, :N] + ps_ref[1, :N]
    c = jnp.maximum(pc_ref[0, :N] + pc_ref[1, :N], 1.0)
    agg = s / c
    v = (jnp.dot(agg, wl_ref[...], preferred_element_type=jnp.float32)
         + bl_ref[...]
         + jnp.dot(h_ref[...], wr_ref[...], preferred_element_type=jnp.float32))
    o_ref[...] = jnp.where(v >= 0, v, 0.01 * v) + h_ref[...]


def _combine_post_body(ps_ref, pc_ref, h_ref, wl_ref, bl_ref, wr_ref,
                       wp_ref, bp_ref, o_ref):
    s = ps_ref[0, :N] + ps_ref[1, :N]
    c = jnp.maximum(pc_ref[0, :N] + pc_ref[1, :N], 1.0)
    agg = s / c
    v = (jnp.dot(agg, wl_ref[...], preferred_element_type=jnp.float32)
         + bl_ref[...]
         + jnp.dot(h_ref[...], wr_ref[...], preferred_element_type=jnp.float32))
    h2 = jnp.where(v >= 0, v, 0.01 * v) + h_ref[...]
    o_ref[...] = jnp.dot(h2, wp_ref[...],
                         preferred_element_type=jnp.float32) + bp_ref[...]


_NH = jax.ShapeDtypeStruct((N, H), jnp.float32)

_tc_pre = pl.pallas_call(_pre_body, out_shape=_NH)
_tc_combine = pl.pallas_call(_combine_body, out_shape=_NH)
_tc_combine_post = pl.pallas_call(_combine_post_body, out_shape=_NH)


def kernel(x, edge_index, W_pre, b_pre, Wl0, bl0, Wr0, Wl1, bl1, Wr1,
           W_post, b_post):
    src = edge_index[0]
    dst = edge_index[1].reshape(NW, NCHUNK, CHUNK)
    zrow = jnp.zeros((RPT, H), jnp.float32)
    zcnt = jnp.zeros((RPT,), jnp.float32)
    ones = jnp.ones((CHUNK,), jnp.float32)

    agg_c, agg_n = _get_sc_aggs()
    h0 = _tc_pre(x, W_pre, b_pre.reshape(1, H))
    ps0, pc0 = agg_c(h0, src, dst, zrow, zcnt, ones)
    pc = pc0.reshape(NC, NP, 1)
    h1 = _tc_combine(ps0, pc, h0, Wl0, bl0.reshape(1, H), Wr0)
    ps1, _ = agg_n(h1, src, dst, zrow, zcnt, ones)
    return _tc_combine_post(ps1, pc, h1, Wl1, bl1.reshape(1, H), Wr1,
                            W_post, b_post.reshape(1, H))


# speedup vs baseline: 2.8415x; 1.0293x over previous
import functools

import jax
import jax.numpy as jnp
from jax import lax
from jax.experimental import pallas as pl
from jax.experimental.pallas import tpu as pltpu
from jax.experimental.pallas import tpu_sc as plsc

N = 10000
E = 320000
H = 128

NC = 2
NS = 16
NW = NC * NS
EPW = E // NW
CHUNK = 80
NCHUNK = EPW // CHUNK
NPAIR = NCHUNK // 2
NP = 10240
RPT = NP // NS
ZROWS = RPT


def _sc_agg_body(with_counts, t_hbm, src_hbm, dst_hbm, zrow_hbm, zcnt_hbm,
                 one_hbm, *rest):
    if with_counts:
        (out_sum, out_cnt, acc, acc_cnt, srcv, dstv, onev, rows_a, rows_b,
         gs_a, gs_b, gs_z, gs_i, gs_c) = rest
    else:
        (out_sum, acc, srcv, dstv, rows_a, rows_b, gs_a, gs_b, gs_z,
         gs_i) = rest
    cid = lax.axis_index("c")
    sid = lax.axis_index("s")
    wid = cid * NS + sid

    zrow_dst = acc.at[pl.ds(sid * RPT, RPT), :]
    pltpu.async_copy(zrow_hbm, zrow_dst, gs_z)
    zcnt_dst = None
    if with_counts:
        zcnt_dst = acc_cnt.at[pl.ds(sid * RPT, RPT)]
        pltpu.async_copy(zcnt_hbm, zcnt_dst, gs_z)
        pltpu.sync_copy(one_hbm, onev)

    src_slab = src_hbm.at[pl.ds(pl.multiple_of(wid * EPW, 8), EPW)]
    pltpu.async_copy(src_slab, srcv, gs_i)
    pltpu.async_copy(dst_hbm.at[wid], dstv, gs_i)
    pltpu.make_async_copy(src_slab, srcv, gs_i).wait()
    pltpu.make_async_copy(dst_hbm.at[wid], dstv, gs_i).wait()
    pltpu.async_copy(t_hbm.at[srcv.at[pl.ds(0, CHUNK)]], rows_a, gs_a)

    pltpu.make_async_copy(zrow_hbm, zrow_dst, gs_z).wait()
    if with_counts:
        pltpu.make_async_copy(zcnt_hbm, zcnt_dst, gs_z).wait()
    plsc.subcore_barrier()

    def gather(c, buf, sem):
        idx = srcv.at[pl.ds(c * CHUNK, CHUNK)]
        return pltpu.async_copy(t_hbm.at[idx], buf, sem)

    def gwait(buf, sem):
        pltpu.make_async_copy(t_hbm.at[srcv.at[pl.ds(0, CHUNK)]],
                              buf, sem).wait()

    def scat(c, buf):
        pltpu.sync_copy(buf, acc.at[dstv.at[c]], add=True)
        if with_counts:
            pltpu.async_copy(onev, acc_cnt.at[dstv.at[c]], add=True,
                             sem=gs_c)

    def pair(g, carry):
        c0 = g * 2
        gather(c0 + 1, rows_b, gs_b)
        gwait(rows_a, gs_a)
        scat(c0, rows_a)
        gather(c0 + 2, rows_a, gs_a)
        gwait(rows_b, gs_b)
        scat(c0 + 1, rows_b)
        return carry

    lax.fori_loop(0, NPAIR, pair, 0)
    gwait(rows_a, gs_a)
    scat(NCHUNK - 1, rows_a)
    if with_counts:
        pltpu.make_async_copy(src_slab, srcv, gs_c).wait()
    plsc.subcore_barrier()

    pltpu.sync_copy(acc.at[pl.ds(sid * RPT, RPT), :],
                    out_sum.at[cid, pl.ds(sid * RPT, RPT), :])
    if with_counts:
        pltpu.sync_copy(acc_cnt.at[pl.ds(sid * RPT, RPT)],
                        out_cnt.at[cid, pl.ds(sid * RPT, RPT)])


_SUM_T = jax.ShapeDtypeStruct((NC, NP, H), jnp.float32)
_CNT_T = jax.ShapeDtypeStruct((NC, NP), jnp.float32)

_SCRATCH_COMMON = dict(
    acc=pltpu.VMEM_SHARED((NP, H), jnp.float32),
    srcv=pltpu.VMEM((EPW,), jnp.int32),
    dstv=pltpu.VMEM((NCHUNK, CHUNK), jnp.int32),
    rows_a=pltpu.VMEM((CHUNK, H), jnp.float32),
    rows_b=pltpu.VMEM((CHUNK, H), jnp.float32),
)

_SCRATCH_C = [
    _SCRATCH_COMMON["acc"],
    pltpu.VMEM_SHARED((NP,), jnp.float32),
    _SCRATCH_COMMON["srcv"],
    _SCRATCH_COMMON["dstv"],
    pltpu.VMEM((CHUNK,), jnp.float32),
    _SCRATCH_COMMON["rows_a"],
    _SCRATCH_COMMON["rows_b"],
    pltpu.SemaphoreType.DMA,
    pltpu.SemaphoreType.DMA,
    pltpu.SemaphoreType.DMA,
    pltpu.SemaphoreType.DMA,
    pltpu.SemaphoreType.DMA,
]

@functools.cache
def _get_sc_aggs():
    mesh = plsc.VectorSubcoreMesh(
        core_axis_name="c", subcore_axis_name="s",
        num_cores=NC, num_subcores=NS)
    agg_c = pl.kernel(functools.partial(_sc_agg_body, True),
                      out_type=(_SUM_T, _CNT_T), mesh=mesh,
                      scratch_types=_SCRATCH_C)
    return agg_c, agg_c


def _pre_body(x_ref, w_ref, b_ref, o_ref):
    v = jnp.dot(x_ref[...], w_ref[...],
                preferred_element_type=jnp.float32) + b_ref[...]
    o_ref[...] = jnp.where(v >= 0, v, 0.01 * v)


def _combine_body(ps_ref, pc_ref, h_ref, wl_ref, bl_ref, wr_ref, o_ref):
    s = ps_ref[0, :N] + ps_ref[1, :N]
    c = jnp.maximum(pc_ref[0, :N] + pc_ref[1, :N], 1.0)
    agg = s / c
    v = (jnp.dot(agg, wl_ref[...], preferred_element_type=jnp.float32)
         + bl_ref[...]
         + jnp.dot(h_ref[...], wr_ref[...], preferred_element_type=jnp.float32))
    o_ref[...] = jnp.where(v >= 0, v, 0.01 * v) + h_ref[...]


def _combine_post_body(ps_ref, pc_ref, h_ref, wl_ref, bl_ref, wr_ref,
                       wp_ref, bp_ref, o_ref):
    s = ps_ref[0, :N] + ps_ref[1, :N]
    c = jnp.maximum(pc_ref[0, :N] + pc_ref[1, :N], 1.0)
    agg = s / c
    v = (jnp.dot(agg, wl_ref[...], preferred_element_type=jnp.float32)
         + bl_ref[...]
         + jnp.dot(h_ref[...], wr_ref[...], preferred_element_type=jnp.float32))
    h2 = jnp.where(v >= 0, v, 0.01 * v) + h_ref[...]
    o_ref[...] = jnp.dot(h2, wp_ref[...],
                         preferred_element_type=jnp.float32) + bp_ref[...]


_NH = jax.ShapeDtypeStruct((N, H), jnp.float32)

_tc_pre = pl.pallas_call(_pre_body, out_shape=_NH)
_tc_combine = pl.pallas_call(_combine_body, out_shape=_NH)
_tc_combine_post = pl.pallas_call(_combine_post_body, out_shape=_NH)


def kernel(x, edge_index, W_pre, b_pre, Wl0, bl0, Wr0, Wl1, bl1, Wr1,
           W_post, b_post):
    src = edge_index[0]
    dst = edge_index[1].reshape(NW, NCHUNK, CHUNK)
    zrow = jnp.zeros((RPT, H), jnp.float32)
    zcnt = jnp.zeros((RPT,), jnp.float32)
    ones = jnp.ones((CHUNK,), jnp.float32)

    agg_c, agg_n = _get_sc_aggs()
    h0 = _tc_pre(x, W_pre, b_pre.reshape(1, H))
    ps0, pc0 = agg_c(h0, src, dst, zrow, zcnt, ones)
    pc = pc0.reshape(NC, NP, 1)
    h1 = _tc_combine(ps0, pc, h0, Wl0, bl0.reshape(1, H), Wr0)
    ps1, _ = agg_n(h1, src, dst, zrow, zcnt, ones)
    return _tc_combine_post(ps1, pc, h1, Wl1, bl1.reshape(1, H), Wr1,
                            W_post, b_post.reshape(1, H))
